# baseline (speedup 1.0000x reference)
import jax
import jax.numpy as jnp
from jax.experimental import pallas as pl

_BB = 32


def _mm_kernel(x_ref, w_ref, o_ref):
    bb, s, k = x_ref.shape
    x = x_ref[...].reshape(bb * s, k)
    o = jax.lax.dot_general(
        x, w_ref[...],
        dimension_numbers=(((1,), (0,)), ((), ())),
        preferred_element_type=jnp.float32,
    )
    o_ref[...] = o.reshape(bb, s, -1)


def kernel(inputs, embeddingRST):
    B, S, K = inputs.shape
    N = embeddingRST.shape[1]
    return pl.pallas_call(
        _mm_kernel,
        grid=(B // _BB,),
        in_specs=[
            pl.BlockSpec((_BB, S, K), lambda i: (i, 0, 0)),
            pl.BlockSpec((K, N), lambda i: (0, 0)),
        ],
        out_specs=pl.BlockSpec((_BB, S, N), lambda i: (i, 0, 0)),
        out_shape=jax.ShapeDtypeStruct((B, S, N), jnp.float32),
    )(inputs, embeddingRST)
